# baseline (speedup 1.0000x reference)
import jax
import jax.numpy as jnp
from jax.experimental import pallas as pl
from jax.experimental.pallas import tpu as pltpu

N = 4096
F = 64
BM = 512
NB = N // BM


def _sgconv_kernel(x_ref, adj_ref, out_ref, adjbf, hx, h1s):
    t = pl.program_id(0)
    even = jax.lax.rem(t, 2) == 0

    @pl.when(t == 0)
    def _init():
        hx[:, 0:F] = jnp.zeros((N, F), jnp.bfloat16)
        hx[:, F:2 * F] = x_ref[...]

    @pl.when(jnp.logical_and(t >= 2, even))
    def _column_pair():
        out_ref[...] = out_ref[...] + jnp.dot(
            adjbf[:, pl.ds((t - 2) * BM, 2 * BM)], h1s[...],
            preferred_element_type=jnp.float32)

        @pl.when(t < NB)
        def _publish():
            hx[pl.ds((t - 2) * BM, 2 * BM), 0:F] = h1s[...]

    @pl.when(t < NB)
    def _sweep():
        abf = adj_ref[...].astype(jnp.bfloat16)
        adjbf[pl.ds(t * BM, BM), :] = abf
        r = jnp.dot(abf, hx[...], preferred_element_type=jnp.float32)
        out_ref[pl.ds(t * BM, BM), :] = r[:, 0:F]
        h1s[pl.ds(jax.lax.rem(t, 2) * BM, BM), :] = (
            r[:, F:2 * F].astype(jnp.bfloat16))


@jax.jit
def kernel(x, adj):
    return pl.pallas_call(
        _sgconv_kernel,
        grid=(NB + 1,),
        in_specs=[
            pl.BlockSpec((N, F), lambda t: (0, 0)),
            pl.BlockSpec((BM, N), lambda t: (jnp.minimum(t, NB - 1), 0)),
        ],
        out_specs=pl.BlockSpec((N, F), lambda t: (0, 0)),
        out_shape=jax.ShapeDtypeStruct((N, F), jnp.float32),
        scratch_shapes=[
            pltpu.VMEM((N, N), jnp.bfloat16),
            pltpu.VMEM((N, 2 * F), jnp.bfloat16),
            pltpu.VMEM((2 * BM, F), jnp.bfloat16),
        ],
    )(x.astype(jnp.bfloat16), adj)


# speedup vs baseline: 1.1179x; 1.1179x over previous
import jax
import jax.numpy as jnp
from jax.experimental import pallas as pl
from jax.experimental.pallas import tpu as pltpu

N = 4096
F = 64
BM = 512
NB = N // BM


def _sgconv_kernel(x_ref, adj_ref, out_ref, adjbf, hx, h1s, acc):
    t = pl.program_id(0)

    @pl.when(t == 0)
    def _init():
        hx[:, 0:F] = jnp.zeros((N, F), jnp.bfloat16)
        hx[:, F:2 * F] = x_ref[...]

    @pl.when(t > 0)
    def _column():
        acc[...] = (acc[...].astype(jnp.float32) + jnp.dot(
            adjbf[:, pl.ds((t - 1) * BM, BM)], h1s[...],
            preferred_element_type=jnp.float32)).astype(jnp.bfloat16)

        @pl.when(t < NB)
        def _publish():
            hx[pl.ds((t - 1) * BM, BM), 0:F] = h1s[...]

    @pl.when(t < NB)
    def _sweep():
        abf = adj_ref[...].astype(jnp.bfloat16)
        adjbf[pl.ds(t * BM, BM), :] = abf
        r = jnp.dot(abf, hx[...], preferred_element_type=jnp.float32)
        acc[pl.ds(t * BM, BM), :] = r[:, 0:F].astype(jnp.bfloat16)
        h1s[...] = r[:, F:2 * F].astype(jnp.bfloat16)

    @pl.when(t == NB)
    def _emit():
        out_ref[...] = acc[...].astype(jnp.float32)


@jax.jit
def kernel(x, adj):
    return pl.pallas_call(
        _sgconv_kernel,
        grid=(NB + 1,),
        in_specs=[
            pl.BlockSpec((N, F), lambda t: (0, 0)),
            pl.BlockSpec((BM, N), lambda t: (jnp.minimum(t, NB - 1), 0)),
        ],
        out_specs=pl.BlockSpec((N, F), lambda t: (0, 0)),
        out_shape=jax.ShapeDtypeStruct((N, F), jnp.float32),
        scratch_shapes=[
            pltpu.VMEM((N, N), jnp.bfloat16),
            pltpu.VMEM((N, 2 * F), jnp.bfloat16),
            pltpu.VMEM((BM, F), jnp.bfloat16),
            pltpu.VMEM((N, F), jnp.bfloat16),
        ],
    )(x.astype(jnp.bfloat16), adj)


# speedup vs baseline: 1.1207x; 1.0025x over previous
import jax
import jax.numpy as jnp
from jax.experimental import pallas as pl
from jax.experimental.pallas import tpu as pltpu

N = 4096
F = 64
BM = 512
NB = N // BM


def _sgconv_kernel(x_ref, adj_ref, out_ref, adjbf, hx, h1s):
    t = pl.program_id(0)

    @pl.when(t == 0)
    def _init():
        hx[:, 0:F] = jnp.zeros((N, F), jnp.bfloat16)
        hx[:, F:2 * F] = x_ref[...]

    @pl.when(t > 0)
    def _column():
        out_ref[...] = out_ref[...] + jnp.dot(
            adjbf[:, pl.ds((t - 1) * BM, BM)], h1s[...],
            preferred_element_type=jnp.float32)
        hx[pl.ds((t - 1) * BM, BM), 0:F] = h1s[...]

    abf = adj_ref[...].astype(jnp.bfloat16)
    adjbf[pl.ds(t * BM, BM), :] = abf
    r = jnp.dot(abf, hx[...], preferred_element_type=jnp.float32)
    out_ref[pl.ds(t * BM, BM), :] = r[:, 0:F]
    h1s[...] = r[:, F:2 * F].astype(jnp.bfloat16)

    @pl.when(t == NB - 1)
    def _last_column():
        out_ref[...] = out_ref[...] + jnp.dot(
            adjbf[:, pl.ds((NB - 1) * BM, BM)],
            r[:, F:2 * F].astype(jnp.bfloat16),
            preferred_element_type=jnp.float32)


@jax.jit
def kernel(x, adj):
    return pl.pallas_call(
        _sgconv_kernel,
        grid=(NB,),
        in_specs=[
            pl.BlockSpec((N, F), lambda t: (0, 0)),
            pl.BlockSpec((BM, N), lambda t: (t, 0)),
        ],
        out_specs=pl.BlockSpec((N, F), lambda t: (0, 0)),
        out_shape=jax.ShapeDtypeStruct((N, F), jnp.float32),
        scratch_shapes=[
            pltpu.VMEM((N, N), jnp.bfloat16),
            pltpu.VMEM((N, 2 * F), jnp.bfloat16),
            pltpu.VMEM((BM, F), jnp.bfloat16),
        ],
    )(x.astype(jnp.bfloat16), adj)


# speedup vs baseline: 1.1258x; 1.0045x over previous
import jax
import jax.numpy as jnp
from jax.experimental import pallas as pl
from jax.experimental.pallas import tpu as pltpu

N = 4096
F = 64
BM = 512
NB = N // BM


def _sgconv_kernel(x_ref, adj_ref, out_ref, adjbf, hx, h1s):
    t = pl.program_id(0)

    @pl.when(t == 0)
    def _init():
        hx[:, 0:F] = jnp.zeros((N, F), jnp.bfloat16)
        hx[:, F:2 * F] = x_ref[...]

    @pl.when(t > 0)
    def _column():
        out_ref[...] = out_ref[...] + jnp.dot(
            adjbf[:, pl.ds((t - 1) * BM, BM)], h1s[...],
            preferred_element_type=jnp.float32)

        @pl.when(t < NB)
        def _publish():
            hx[pl.ds((t - 1) * BM, BM), 0:F] = h1s[...]

    @pl.when(t < NB)
    def _sweep():
        abf = adj_ref[...].astype(jnp.bfloat16)
        adjbf[pl.ds(t * BM, BM), :] = abf
        r = jnp.dot(abf, hx[...], preferred_element_type=jnp.float32)
        out_ref[pl.ds(t * BM, BM), :] = r[:, 0:F]
        h1s[...] = r[:, F:2 * F].astype(jnp.bfloat16)


@jax.jit
def kernel(x, adj):
    return pl.pallas_call(
        _sgconv_kernel,
        grid=(NB + 1,),
        in_specs=[
            pl.BlockSpec((N, F), lambda t: (0, 0)),
            pl.BlockSpec((BM, N), lambda t: (jnp.minimum(t, NB - 1), 0)),
        ],
        out_specs=pl.BlockSpec((N, F), lambda t: (0, 0)),
        out_shape=jax.ShapeDtypeStruct((N, F), jnp.float32),
        scratch_shapes=[
            pltpu.VMEM((N, N), jnp.bfloat16),
            pltpu.VMEM((N, 2 * F), jnp.bfloat16),
            pltpu.VMEM((BM, F), jnp.bfloat16),
        ],
    )(x.astype(jnp.bfloat16), adj)
